# baseline (speedup 1.0000x reference)
import dataclasses
import functools

import jax
import jax.numpy as jnp
from jax import lax
from jax.experimental import pallas as pl
from jax.experimental.pallas import tpu as pltpu
from jax.experimental.pallas import tpu_sc as plsc

N = 10000
E = 160000
D = 256
DH = 128

NC = 2
NS = 16
CHUNK = 128


def _make_sc_aggregate(Nn, Ee):
  EPT = Ee // NS
  NFULL = EPT // CHUNK
  REM = EPT - NFULL * CHUNK
  DEG_SPLIT = NFULL // 2
  RPT = (Nn // NS) // 8 * 8
  TAIL = Nn - NS * RPT
  NR = (-(-Nn // 128) + 7) // 8 * 8

  mesh = plsc.VectorSubcoreMesh(core_axis_name="c", subcore_axis_name="s")
  out_type = (
      jax.ShapeDtypeStruct((Nn, DH), jnp.float32),
      jax.ShapeDtypeStruct((Nn, DH), jnp.float32),
      jax.ShapeDtypeStruct((NR, 128), jnp.float32),
      jax.ShapeDtypeStruct((NR, 128), jnp.float32),
  )
  cp = pltpu.CompilerParams()
  if "needs_layout_passes" in pltpu.CompilerParams.__dataclass_fields__:
    cp = dataclasses.replace(cp, needs_layout_passes=False)

  @functools.partial(
      pl.kernel,
      mesh=mesh,
      out_type=out_type,
      compiler_params=cp,
      scratch_types=[
          pltpu.VMEM_SHARED((Nn, DH), jnp.float32),
          pltpu.VMEM_SHARED((NR, 128), jnp.float32),
          pltpu.VMEM((CHUNK,), jnp.int32),
          pltpu.VMEM((CHUNK,), jnp.int32),
          pltpu.VMEM((CHUNK,), jnp.int32),
          pltpu.VMEM((CHUNK,), jnp.int32),
          pltpu.VMEM((CHUNK, DH), jnp.float32),
          pltpu.VMEM((CHUNK, DH), jnp.float32),
          pltpu.VMEM((max(REM, 8),), jnp.int32),
          pltpu.VMEM((max(REM, 8), DH), jnp.float32),
          pltpu.VMEM((NR, 128), jnp.float32),
          pltpu.VMEM((NR,), jnp.int32),
          pltpu.SemaphoreType.DMA,
          pltpu.SemaphoreType.DMA,
          pltpu.SemaphoreType.DMA,
          pltpu.SemaphoreType.DMA,
          pltpu.SemaphoreType.DMA,
      ],
  )
  def k(xcat_hbm, src_hbm, dst_hbm, zeros_hbm, iota_hbm,
        agg0_hbm, agg1_hbm, deg0_hbm, deg1_hbm,
        acc, dacc, src_v0, src_v1, dst_v0, dst_v1, rows_v0, rows_v1,
        dst_r, rows_r, hist, iota_v, sem, gsem0, gsem1, isem0, isem1):
    c = lax.axis_index("c")
    t = lax.axis_index("s")
    row0 = t * RPT
    pltpu.async_copy(
        xcat_hbm.at[pl.ds(c * Nn + row0, RPT)], acc.at[pl.ds(row0, RPT)], sem
    ).wait()
    if TAIL:
      @pl.when(t == NS - 1)
      def _():
        pltpu.async_copy(
            xcat_hbm.at[pl.ds(c * Nn + NS * RPT, TAIL)],
            acc.at[pl.ds(NS * RPT, TAIL)], sem
        ).wait()

    ebase = t * EPT
    sbase = c * Ee + ebase
    pltpu.async_copy(zeros_hbm.at[pl.ds(0, NR)], hist, sem).wait()
    pltpu.async_copy(iota_hbm, iota_v, sem).wait()

    @pl.when(t == 0)
    def _():
      pltpu.sync_copy(zeros_hbm.at[pl.ds(0, NR)], dacc)

    plsc.subcore_barrier()

    ones16 = jnp.ones((16,), jnp.float32)

    def count(idx):
      plsc.addupdate_scatter(
          hist, [lax.shift_right_logical(idx, 7), lax.bitwise_and(idx, 127)],
          ones16)

    src_slots = (src_v0, src_v1)
    dst_slots = (dst_v0, dst_v1)
    row_slots = (rows_v0, rows_v1)
    gsems = (gsem0, gsem1)
    isems = (isem0, isem1)

    def idx_load(i, w):
      pltpu.async_copy(src_hbm.at[pl.ds(sbase + i * CHUNK, CHUNK)],
                       src_slots[w], isems[w])
      pltpu.async_copy(dst_hbm.at[pl.ds(ebase + i * CHUNK, CHUNK)],
                       dst_slots[w], isems[w])

    def idx_wait(w):
      pltpu.make_async_copy(src_hbm.at[pl.ds(0, CHUNK)],
                            src_slots[w], isems[w]).wait()
      pltpu.make_async_copy(dst_hbm.at[pl.ds(0, CHUNK)],
                            dst_slots[w], isems[w]).wait()

    def gather(i, w):
      pltpu.async_copy(xcat_hbm.at[src_slots[w]], row_slots[w], gsems[w])

    idx_load(0, 0)
    idx_load(1, 1)
    idx_wait(0)
    gather(0, 0)

    @pl.loop(0, NFULL, step=2)
    def _(kk):
      for u in (0, 1):
        i = kk + u
        w = 1 - u
        if u == 0:
          idx_wait(w)
          gather(i + 1, w)
        else:
          @pl.when(i + 1 < NFULL)
          def _():
            idx_wait(w)
            gather(i + 1, w)
        pltpu.make_async_copy(
            xcat_hbm.at[src_slots[u]], row_slots[u], gsems[u]).wait()
        pltpu.sync_copy(row_slots[u], acc.at[dst_slots[u]], add=True)

        @pl.when((c == 0) == (i < DEG_SPLIT))
        def _():
          @pl.loop(0, CHUNK, step=16)
          def _(j):
            count(dst_slots[u][pl.ds(j, 16)])

        if u == 0:
          @pl.when(i + 2 < NFULL)
          def _():
            idx_load(i + 2, u)
        else:
          @pl.when(i + 2 < NFULL)
          def _():
            idx_load(i + 2, u)

    if REM:
      off = ebase + NFULL * CHUNK
      pltpu.sync_copy(dst_hbm.at[pl.ds(off, REM)], dst_r)
      pltpu.sync_copy(src_hbm.at[pl.ds(sbase + NFULL * CHUNK, REM)],
                      src_v0.at[pl.ds(0, REM)])
      pltpu.async_copy(
          xcat_hbm.at[src_v0.at[pl.ds(0, REM)]], rows_r, sem).wait()
      pltpu.sync_copy(rows_r, acc.at[dst_r], add=True)

      @pl.when(c == 1)
      def _():
        @pl.loop(0, REM, step=16)
        def _(j):
          count(dst_r[pl.ds(j, 16)])

    pltpu.sync_copy(hist, dacc.at[iota_v], add=True)
    plsc.subcore_barrier()

    @pl.when(c == 0)
    def _():
      @pl.when(t == 0)
      def _():
        pltpu.sync_copy(dacc, deg0_hbm)
      pltpu.sync_copy(acc.at[pl.ds(row0, RPT)], agg0_hbm.at[pl.ds(row0, RPT)])
      if TAIL:
        @pl.when(t == NS - 1)
        def _():
          pltpu.sync_copy(acc.at[pl.ds(NS * RPT, TAIL)],
                          agg0_hbm.at[pl.ds(NS * RPT, TAIL)])

    @pl.when(c == 1)
    def _():
      @pl.when(t == 0)
      def _():
        pltpu.sync_copy(dacc, deg1_hbm)
      pltpu.sync_copy(acc.at[pl.ds(row0, RPT)], agg1_hbm.at[pl.ds(row0, RPT)])
      if TAIL:
        @pl.when(t == NS - 1)
        def _():
          pltpu.sync_copy(acc.at[pl.ds(NS * RPT, TAIL)],
                          agg1_hbm.at[pl.ds(NS * RPT, TAIL)])

  return k, NR, RPT


BR = 1000


def _tc_body(agg0, agg1, deg0, deg1, w, b, gamma, beta, out):
  deg = deg0[...] + deg1[...] + 1.0
  h = jnp.concatenate([agg0[...], agg1[...]], axis=1) / deg
  y = lax.dot_general(
      h, w[...], (((1,), (1,)), ((), ())), preferred_element_type=jnp.float32
  ) + b[...]
  mean = jnp.mean(y, axis=1, keepdims=True)
  var = jnp.mean((y - mean) ** 2, axis=1, keepdims=True)
  yn = (y - mean) * lax.rsqrt(var + 1e-5) * gamma[...] + beta[...]
  out[...] = jnp.maximum(yn, 0.0)


def _tc_linear_ln_relu(agg0, agg1, deg0, deg1, W, b, gamma, beta):
  grid = (N // BR,)
  return pl.pallas_call(
      _tc_body,
      grid=grid,
      in_specs=[
          pl.BlockSpec((BR, DH), lambda i: (i, 0)),
          pl.BlockSpec((BR, DH), lambda i: (i, 0)),
          pl.BlockSpec((BR, 1), lambda i: (i, 0)),
          pl.BlockSpec((BR, 1), lambda i: (i, 0)),
          pl.BlockSpec((D, D), lambda i: (0, 0)),
          pl.BlockSpec((1, D), lambda i: (0, 0)),
          pl.BlockSpec((1, D), lambda i: (0, 0)),
          pl.BlockSpec((1, D), lambda i: (0, 0)),
      ],
      out_specs=pl.BlockSpec((BR, D), lambda i: (i, 0)),
      out_shape=jax.ShapeDtypeStruct((N, D), jnp.float32),
  )(agg0, agg1, deg0, deg1, W, b, gamma, beta)


def kernel(x, edge_index, W, b, gamma, beta):
  src = edge_index[0].astype(jnp.int32)
  dst = edge_index[1].astype(jnp.int32)
  xcat = jnp.concatenate([x[:, :DH], x[:, DH:]], axis=0)
  srcadj = jnp.concatenate([src, src + N])
  sc_fn, NR, RPT = _make_sc_aggregate(N, E)
  zeros = jnp.zeros((RPT, 128), jnp.float32)
  iota = jnp.arange(NR, dtype=jnp.int32)
  agg0, agg1, deg0, deg1 = sc_fn(xcat, srcadj, dst, zeros, iota)
  return _tc_linear_ln_relu(
      agg0, agg1,
      deg0.reshape(-1)[:N].reshape(N, 1),
      deg1.reshape(-1)[:N].reshape(N, 1), W,
      b.reshape(1, D), gamma.reshape(1, D), beta.reshape(1, D)
  )


# speedup vs baseline: 1.0069x; 1.0069x over previous
import dataclasses
import functools

import jax
import jax.numpy as jnp
from jax import lax
from jax.experimental import pallas as pl
from jax.experimental.pallas import tpu as pltpu
from jax.experimental.pallas import tpu_sc as plsc

N = 10000
E = 160000
D = 256
DH = 128

NC = 2
NS = 16
CHUNK = 128


def _make_sc_aggregate(Nn, Ee):
  EPT = Ee // NS
  NFULL = EPT // CHUNK
  REM = EPT - NFULL * CHUNK
  DEG_SPLIT = NFULL // 2
  RPT = (Nn // NS) // 8 * 8
  TAIL = Nn - NS * RPT
  NR = (-(-Nn // 128) + 7) // 8 * 8

  mesh = plsc.VectorSubcoreMesh(core_axis_name="c", subcore_axis_name="s")
  out_type = (
      jax.ShapeDtypeStruct((Nn, DH), jnp.float32),
      jax.ShapeDtypeStruct((Nn, DH), jnp.float32),
      jax.ShapeDtypeStruct((NR, 128), jnp.float32),
      jax.ShapeDtypeStruct((NR, 128), jnp.float32),
  )
  cp = pltpu.CompilerParams()
  if "needs_layout_passes" in pltpu.CompilerParams.__dataclass_fields__:
    cp = dataclasses.replace(cp, needs_layout_passes=False)

  @functools.partial(
      pl.kernel,
      mesh=mesh,
      out_type=out_type,
      compiler_params=cp,
      scratch_types=[
          pltpu.VMEM_SHARED((Nn, DH), jnp.float32),
          pltpu.VMEM_SHARED((NR, 128), jnp.float32),
          pltpu.VMEM((CHUNK,), jnp.int32),
          pltpu.VMEM((CHUNK,), jnp.int32),
          pltpu.VMEM((CHUNK,), jnp.int32),
          pltpu.VMEM((CHUNK,), jnp.int32),
          pltpu.VMEM((CHUNK, DH), jnp.float32),
          pltpu.VMEM((CHUNK, DH), jnp.float32),
          pltpu.VMEM((max(REM, 8),), jnp.int32),
          pltpu.VMEM((max(REM, 8), DH), jnp.float32),
          pltpu.VMEM((NR, 128), jnp.float32),
          pltpu.VMEM((NR,), jnp.int32),
          pltpu.SemaphoreType.DMA,
          pltpu.SemaphoreType.DMA,
          pltpu.SemaphoreType.DMA,
          pltpu.SemaphoreType.DMA,
          pltpu.SemaphoreType.DMA,
      ],
  )
  def k(xcat_hbm, src_hbm, dst_hbm, zeros_hbm, iota_hbm,
        agg0_hbm, agg1_hbm, deg0_hbm, deg1_hbm,
        acc, dacc, src_v0, src_v1, dst_v0, dst_v1, rows_v0, rows_v1,
        dst_r, rows_r, hist, iota_v, sem, gsem0, gsem1, isem0, isem1):
    c = lax.axis_index("c")
    t = lax.axis_index("s")
    row0 = t * RPT
    pltpu.async_copy(
        xcat_hbm.at[pl.ds(c * Nn + row0, RPT)], acc.at[pl.ds(row0, RPT)], sem
    ).wait()
    if TAIL:
      @pl.when(t == NS - 1)
      def _():
        pltpu.async_copy(
            xcat_hbm.at[pl.ds(c * Nn + NS * RPT, TAIL)],
            acc.at[pl.ds(NS * RPT, TAIL)], sem
        ).wait()

    ebase = t * EPT
    sbase = c * Ee + ebase
    pltpu.async_copy(zeros_hbm.at[pl.ds(0, NR)], hist, sem).wait()
    pltpu.async_copy(iota_hbm, iota_v, sem).wait()

    @pl.when(t == 0)
    def _():
      pltpu.sync_copy(zeros_hbm.at[pl.ds(0, NR)], dacc)

    plsc.subcore_barrier()

    ones16 = jnp.ones((16,), jnp.float32)

    def count(idx):
      plsc.addupdate_scatter(
          hist, [lax.shift_right_logical(idx, 7), lax.bitwise_and(idx, 127)],
          ones16)

    src_slots = (src_v0, src_v1)
    dst_slots = (dst_v0, dst_v1)
    row_slots = (rows_v0, rows_v1)
    gsems = (gsem0, gsem1)
    isems = (isem0, isem1)

    def idx_load(i, w):
      pltpu.async_copy(src_hbm.at[pl.ds(sbase + i * CHUNK, CHUNK)],
                       src_slots[w], isems[w])
      pltpu.async_copy(dst_hbm.at[pl.ds(ebase + i * CHUNK, CHUNK)],
                       dst_slots[w], isems[w])

    def idx_wait(w):
      pltpu.make_async_copy(src_hbm.at[pl.ds(0, CHUNK)],
                            src_slots[w], isems[w]).wait()
      pltpu.make_async_copy(dst_hbm.at[pl.ds(0, CHUNK)],
                            dst_slots[w], isems[w]).wait()

    def gather(i, w):
      pltpu.async_copy(xcat_hbm.at[src_slots[w]], row_slots[w], gsems[w])

    idx_load(0, 0)
    idx_load(1, 1)
    idx_wait(0)
    gather(0, 0)

    @pl.loop(0, NFULL, step=2)
    def _(kk):
      for u in (0, 1):
        i = kk + u
        w = 1 - u
        if u == 0:
          idx_wait(w)
          gather(i + 1, w)
        else:
          @pl.when(i + 1 < NFULL)
          def _():
            idx_wait(w)
            gather(i + 1, w)
        @pl.when((c == 0) == (i < DEG_SPLIT))
        def _():
          @pl.loop(0, CHUNK, step=16)
          def _(j):
            count(dst_slots[u][pl.ds(j, 16)])

        pltpu.make_async_copy(
            xcat_hbm.at[src_slots[u]], row_slots[u], gsems[u]).wait()
        pltpu.sync_copy(row_slots[u], acc.at[dst_slots[u]], add=True)

        if u == 0:
          @pl.when(i + 2 < NFULL)
          def _():
            idx_load(i + 2, u)
        else:
          @pl.when(i + 2 < NFULL)
          def _():
            idx_load(i + 2, u)

    if REM:
      off = ebase + NFULL * CHUNK
      pltpu.sync_copy(dst_hbm.at[pl.ds(off, REM)], dst_r)
      pltpu.sync_copy(src_hbm.at[pl.ds(sbase + NFULL * CHUNK, REM)],
                      src_v0.at[pl.ds(0, REM)])
      pltpu.async_copy(
          xcat_hbm.at[src_v0.at[pl.ds(0, REM)]], rows_r, sem).wait()
      pltpu.sync_copy(rows_r, acc.at[dst_r], add=True)

      @pl.when(c == 1)
      def _():
        @pl.loop(0, REM, step=16)
        def _(j):
          count(dst_r[pl.ds(j, 16)])

    pltpu.sync_copy(hist, dacc.at[iota_v], add=True)
    plsc.subcore_barrier()

    @pl.when(c == 0)
    def _():
      @pl.when(t == 0)
      def _():
        pltpu.sync_copy(dacc, deg0_hbm)
      pltpu.sync_copy(acc.at[pl.ds(row0, RPT)], agg0_hbm.at[pl.ds(row0, RPT)])
      if TAIL:
        @pl.when(t == NS - 1)
        def _():
          pltpu.sync_copy(acc.at[pl.ds(NS * RPT, TAIL)],
                          agg0_hbm.at[pl.ds(NS * RPT, TAIL)])

    @pl.when(c == 1)
    def _():
      @pl.when(t == 0)
      def _():
        pltpu.sync_copy(dacc, deg1_hbm)
      pltpu.sync_copy(acc.at[pl.ds(row0, RPT)], agg1_hbm.at[pl.ds(row0, RPT)])
      if TAIL:
        @pl.when(t == NS - 1)
        def _():
          pltpu.sync_copy(acc.at[pl.ds(NS * RPT, TAIL)],
                          agg1_hbm.at[pl.ds(NS * RPT, TAIL)])

  return k, NR, RPT


BR = 1000


def _tc_body(agg0, agg1, deg0, deg1, w, b, gamma, beta, out):
  deg = deg0[...] + deg1[...] + 1.0
  h = jnp.concatenate([agg0[...], agg1[...]], axis=1) / deg
  y = lax.dot_general(
      h, w[...], (((1,), (1,)), ((), ())), preferred_element_type=jnp.float32
  ) + b[...]
  mean = jnp.mean(y, axis=1, keepdims=True)
  var = jnp.mean((y - mean) ** 2, axis=1, keepdims=True)
  yn = (y - mean) * lax.rsqrt(var + 1e-5) * gamma[...] + beta[...]
  out[...] = jnp.maximum(yn, 0.0)


def _tc_linear_ln_relu(agg0, agg1, deg0, deg1, W, b, gamma, beta):
  grid = (N // BR,)
  return pl.pallas_call(
      _tc_body,
      grid=grid,
      in_specs=[
          pl.BlockSpec((BR, DH), lambda i: (i, 0)),
          pl.BlockSpec((BR, DH), lambda i: (i, 0)),
          pl.BlockSpec((BR, 1), lambda i: (i, 0)),
          pl.BlockSpec((BR, 1), lambda i: (i, 0)),
          pl.BlockSpec((D, D), lambda i: (0, 0)),
          pl.BlockSpec((1, D), lambda i: (0, 0)),
          pl.BlockSpec((1, D), lambda i: (0, 0)),
          pl.BlockSpec((1, D), lambda i: (0, 0)),
      ],
      out_specs=pl.BlockSpec((BR, D), lambda i: (i, 0)),
      out_shape=jax.ShapeDtypeStruct((N, D), jnp.float32),
  )(agg0, agg1, deg0, deg1, W, b, gamma, beta)


def kernel(x, edge_index, W, b, gamma, beta):
  src = edge_index[0].astype(jnp.int32)
  dst = edge_index[1].astype(jnp.int32)
  xcat = jnp.concatenate([x[:, :DH], x[:, DH:]], axis=0)
  srcadj = jnp.concatenate([src, src + N])
  sc_fn, NR, RPT = _make_sc_aggregate(N, E)
  zeros = jnp.zeros((RPT, 128), jnp.float32)
  iota = jnp.arange(NR, dtype=jnp.int32)
  agg0, agg1, deg0, deg1 = sc_fn(xcat, srcadj, dst, zeros, iota)
  return _tc_linear_ln_relu(
      agg0, agg1,
      deg0.reshape(-1)[:N].reshape(N, 1),
      deg1.reshape(-1)[:N].reshape(N, 1), W,
      b.reshape(1, D), gamma.reshape(1, D), beta.reshape(1, D)
  )
